# baseline (speedup 1.0000x reference)
import functools

import jax
import jax.numpy as jnp
from jax import lax
from jax.experimental import pallas as pl
from jax.experimental.pallas import tpu as pltpu
from jax.experimental.pallas import tpu_sc as plsc

NC = 2
NS = 16
NW = NC * NS

CB = 4
GW = 100


@functools.partial(jax.jit, static_argnames=("b", "s", "d"))
def _gather(x, table, *, b, s, d):
    batches_per_w = b // NW
    pairs = batches_per_w // (2 * CB)

    @functools.partial(
        pl.kernel,
        out_type=jax.ShapeDtypeStruct((b, s, d), jnp.float32),
        mesh=plsc.VectorSubcoreMesh(core_axis_name="c", subcore_axis_name="s"),
        scratch_types=[
            pltpu.VMEM((2 * CB, s), jnp.int32),
            pltpu.VMEM((CB, s, d), jnp.float32),
            pltpu.VMEM((CB, s, d), jnp.float32),
            pltpu.SemaphoreType.DMA,
            pltpu.SemaphoreType.DMA,
        ],
        compiler_params=pltpu.CompilerParams(use_tc_tiling_on_sc=False),
    )
    def body(x_hbm, table_hbm, out_hbm, idx_v, rows0, rows1, gsem, osem):
        wid = lax.axis_index("s") * NC + lax.axis_index("c")

        def fire_gathers(rows_v, half):
            return [
                pltpu.async_copy(
                    table_hbm.at[idx_v.at[half * CB + bb]],
                    rows_v.at[bb],
                    gsem,
                )
                for bb in range(CB)
            ]

        def store_wait(rows_v):
            pltpu.make_async_copy(rows_v, out_hbm.at[pl.ds(0, CB)], osem).wait()

        def pair_body(g, first):
            b0 = wid * batches_per_w + g * 2 * CB
            pltpu.sync_copy(x_hbm.at[pl.ds(b0, 2 * CB)], idx_v)
            if not first:
                store_wait(rows0)
            ga = fire_gathers(rows0, 0)
            if not first:
                store_wait(rows1)
            for c in ga:
                c.wait()
            gb = fire_gathers(rows1, 1)
            pltpu.async_copy(rows0, out_hbm.at[pl.ds(b0, CB)], osem)
            for c in gb:
                c.wait()
            pltpu.async_copy(rows1, out_hbm.at[pl.ds(b0 + CB, CB)], osem)

        pair_body(0, True)

        def step(g, carry):
            pair_body(g, False)
            return carry

        lax.fori_loop(1, pairs, step, 0)
        store_wait(rows0)
        store_wait(rows1)

    return body(x, table)


P = 4


def kernel(x, table):
    b, s = x.shape
    v, d = table.shape
    x = x.astype(jnp.int32)
    sp = s // P
    outs = [
        _gather(x[:, p * sp:(p + 1) * sp], table, b=b, s=sp, d=d)
        for p in range(P)
    ]
    return jnp.concatenate(outs, axis=1)
